# Initial kernel scaffold; baseline (speedup 1.0000x reference)
import jax
import jax.numpy as jnp
from jax.experimental import pallas as pl


def kernel(x, Wr, br, Wq, Wk, Wv, Wo):
    raise NotImplementedError("write your pallas kernel here")


# speedup vs baseline: 1.4768x; 1.4768x over previous
import functools
import jax
import jax.numpy as jnp
from jax import lax
from jax.experimental import pallas as pl
from jax.experimental.pallas import tpu as pltpu
from jax.experimental.pallas import tpu_sc as plsc

B, L, D = 2, 2048, 2048
H, DH = 16, 128
S = 1024
NEG = -1e30


_RANK_CHUNK = 256


def _router_kernel(x_ref, wr_ref, br_ref, padj_ref, s_ref, rank_ref):
    b = pl.program_id(0)
    s = jnp.dot(
        x_ref[0].astype(jnp.bfloat16),
        wr_ref[...].astype(jnp.bfloat16),
        preferred_element_type=jnp.float32,
    )[:, 0] + br_ref[0]
    s_ref[0, :] = s

    iota_j = lax.broadcasted_iota(jnp.int32, (1, L), 1).astype(jnp.float32)

    def rank_body(c, _):
        base = c * _RANK_CHUNK
        sv = s_ref[0, pl.ds(base, _RANK_CHUNK)].reshape(_RANK_CHUNK, 1)
        row_i = (lax.broadcasted_iota(jnp.int32, (_RANK_CHUNK, 1), 0)
                 + base).astype(jnp.float32)
        srow = s_ref[0, :].reshape(1, L)
        gt = (srow > sv).astype(jnp.float32)
        tie = jnp.logical_and(srow == sv, iota_j < row_i).astype(jnp.float32)
        rank = jnp.sum(gt + tie, axis=1)
        rank_ref[0, pl.ds(base, _RANK_CHUNK)] = rank
        return _

    lax.fori_loop(0, L // _RANK_CHUNK, rank_body, 0)

    def perm_body(c, _):
        base = c * _RANK_CHUNK
        rv = (lax.broadcasted_iota(jnp.int32, (_RANK_CHUNK, 1), 0)
              + base).astype(jnp.float32)
        rrow = rank_ref[0, :].reshape(1, L)
        eq = (rrow == rv).astype(jnp.float32)
        idx = jnp.sum(eq * iota_j, axis=1)
        padj_ref[0, 0, pl.ds(base, _RANK_CHUNK)] = idx.astype(jnp.int32) + b * L
        return _

    lax.fori_loop(0, L // _RANK_CHUNK, perm_body, 0)


def _router_topk(x, Wr, br):
    return pl.pallas_call(
        _router_kernel,
        grid=(B,),
        in_specs=[
            pl.BlockSpec((1, L, D), lambda b: (b, 0, 0)),
            pl.BlockSpec((D, 1), lambda b: (0, 0)),
            pl.BlockSpec((1,), lambda b: (0,), memory_space=pltpu.SMEM),
        ],
        out_specs=pl.BlockSpec((1, 1, L), lambda b: (b, 0, 0)),
        out_shape=jax.ShapeDtypeStruct((B, 1, L), jnp.int32),
        scratch_shapes=[
            pltpu.VMEM((1, L), jnp.float32),
            pltpu.VMEM((1, L), jnp.float32),
        ],
    )(x, Wr, br)


_QKV_BM = 512
_QKV_BN = 512


def _qkv_kernel(xs_ref, wq_ref, wk_ref, wv_ref, q_ref, k_ref, v_ref):
    xb = xs_ref[0].astype(jnp.bfloat16)
    for w_ref, o_ref in ((wq_ref, q_ref), (wk_ref, k_ref), (wv_ref, v_ref)):
        wb = w_ref[...].astype(jnp.bfloat16)
        acc = jnp.dot(xb, wb, preferred_element_type=jnp.float32)
        o_ref[0] = acc.astype(jnp.bfloat16)


def _qkv(xs3, Wq, Wk, Wv):
    grid = (D // _QKV_BN, B, S // _QKV_BM)
    return pl.pallas_call(
        _qkv_kernel,
        grid=grid,
        in_specs=[
            pl.BlockSpec((1, _QKV_BM, D), lambda n, b, m: (b, m, 0)),
            pl.BlockSpec((D, _QKV_BN), lambda n, b, m: (0, n)),
            pl.BlockSpec((D, _QKV_BN), lambda n, b, m: (0, n)),
            pl.BlockSpec((D, _QKV_BN), lambda n, b, m: (0, n)),
        ],
        out_specs=[
            pl.BlockSpec((1, _QKV_BM, _QKV_BN), lambda n, b, m: (b, m, n)),
            pl.BlockSpec((1, _QKV_BM, _QKV_BN), lambda n, b, m: (b, m, n)),
            pl.BlockSpec((1, _QKV_BM, _QKV_BN), lambda n, b, m: (b, m, n)),
        ],
        out_shape=[jax.ShapeDtypeStruct((B, S, D), jnp.bfloat16)] * 3,
    )(xs3, Wq, Wk, Wv)


_ATT_BQ = 512
_SCALE = 1.0 / float(jnp.sqrt(jnp.float32(DH)))


def _attn_kernel(q_ref, k_ref, v_ref, o_ref):
    qi = pl.program_id(1)
    qpos = lax.broadcasted_iota(jnp.int32, (_ATT_BQ, S), 0) + qi * _ATT_BQ
    kpos = lax.broadcasted_iota(jnp.int32, (_ATT_BQ, S), 1)
    causal = qpos >= kpos
    qb = q_ref[0]
    kb = k_ref[0]
    vb = v_ref[0]
    for h in range(H):
        sl = slice(h * DH, (h + 1) * DH)
        qh = qb[:, sl]
        kh = kb[:, sl]
        vh = vb[:, sl]
        s = lax.dot_general(
            qh, kh, (((1,), (1,)), ((), ())),
            preferred_element_type=jnp.float32,
        ) * _SCALE
        s = jnp.where(causal, s, NEG)
        m = jnp.max(s, axis=1, keepdims=True)
        p = jnp.exp(s - m)
        l = jnp.sum(p, axis=1, keepdims=True)
        o = jnp.dot(p.astype(jnp.bfloat16), vh,
                    preferred_element_type=jnp.float32)
        o_ref[0, :, sl] = (o / l).astype(jnp.bfloat16)


def _attention(q, k, v):
    return pl.pallas_call(
        _attn_kernel,
        grid=(B, S // _ATT_BQ),
        in_specs=[
            pl.BlockSpec((1, _ATT_BQ, D), lambda b, m: (b, m, 0)),
            pl.BlockSpec((1, S, D), lambda b, m: (b, 0, 0)),
            pl.BlockSpec((1, S, D), lambda b, m: (b, 0, 0)),
        ],
        out_specs=pl.BlockSpec((1, _ATT_BQ, D), lambda b, m: (b, m, 0)),
        out_shape=jax.ShapeDtypeStruct((B, S, D), jnp.bfloat16),
    )(q, k, v)


_PRJ_BM = 512
_PRJ_BN = 1024


def _proj_kernel(a_ref, wo_ref, o_ref):
    wb = wo_ref[...].astype(jnp.bfloat16)
    o_ref[0] = jnp.dot(a_ref[0], wb, preferred_element_type=jnp.float32)


def _proj(attn, Wo):
    grid = (D // _PRJ_BN, B, S // _PRJ_BM)
    return pl.pallas_call(
        _proj_kernel,
        grid=grid,
        in_specs=[
            pl.BlockSpec((1, _PRJ_BM, D), lambda n, b, m: (b, m, 0)),
            pl.BlockSpec((D, _PRJ_BN), lambda n, b, m: (0, n)),
        ],
        out_specs=pl.BlockSpec((1, _PRJ_BM, _PRJ_BN), lambda n, b, m: (b, m, n)),
        out_shape=jax.ShapeDtypeStruct((B, S, D), jnp.float32),
    )(attn, Wo)


_NW = 32
_CH = 16

def _sc_mesh():
    return plsc.VectorSubcoreMesh(core_axis_name="c", subcore_axis_name="s")


def _sc_gather(x2d, gidx):
    n = gidx.shape[0]
    rows_w = n // _NW

    @functools.partial(
        pl.kernel,
        out_type=jax.ShapeDtypeStruct((n, D), jnp.float32),
        mesh=_sc_mesh(),
        scratch_types=[
            pltpu.VMEM((_CH,), jnp.int32),
            pltpu.VMEM((_CH, D), jnp.float32),
            pltpu.SemaphoreType.DMA,
        ],
    )
    def k(x_hbm, i_hbm, o_hbm, idx_v, rows_v, sem):
        wid = lax.axis_index("s") * 2 + lax.axis_index("c")
        base = wid * rows_w

        @pl.loop(0, rows_w, step=_CH)
        def _(c):
            pltpu.sync_copy(i_hbm.at[pl.ds(base + c, _CH)], idx_v)
            pltpu.async_copy(x_hbm.at[idx_v], rows_v, sem).wait()
            pltpu.sync_copy(rows_v, o_hbm.at[pl.ds(base + c, _CH)])

    return k(x2d, gidx)


def _sc_combine(x2d, out2d, sidx, cidx):
    ns = sidx.shape[0]
    nc = cidx.shape[0]
    ns_w = ns // _NW
    nc_w = nc // _NW

    @functools.partial(
        pl.kernel,
        out_type=jax.ShapeDtypeStruct((B * L, D), jnp.float32),
        mesh=_sc_mesh(),
        scratch_types=[
            pltpu.VMEM((_CH,), jnp.int32),
            pltpu.VMEM((_CH, D), jnp.float32),
            pltpu.SemaphoreType.DMA,
        ],
    )
    def k(x_hbm, out_hbm, si_hbm, ci_hbm, o_hbm, idx_v, rows_v, sem):
        wid = lax.axis_index("s") * 2 + lax.axis_index("c")
        sbase = wid * ns_w
        cbase = wid * nc_w

        @pl.loop(0, ns_w, step=_CH)
        def _(c):
            pltpu.sync_copy(si_hbm.at[pl.ds(sbase + c, _CH)], idx_v)
            pltpu.sync_copy(out_hbm.at[pl.ds(sbase + c, _CH)], rows_v)
            pltpu.async_copy(rows_v, o_hbm.at[idx_v], sem).wait()

        @pl.loop(0, nc_w, step=_CH)
        def _(c):
            pltpu.sync_copy(ci_hbm.at[pl.ds(cbase + c, _CH)], idx_v)
            pltpu.async_copy(x_hbm.at[idx_v], rows_v, sem).wait()
            pltpu.async_copy(rows_v, o_hbm.at[idx_v], sem).wait()

    return k(x2d, out2d, sidx, cidx)


def kernel(x, Wr, br, Wq, Wk, Wv, Wo):
    padj = _router_topk(x, Wr, br)[:, 0, :]
    sidx = padj[:, :S].reshape(-1)
    cidx = padj[:, S:].reshape(-1)

    x2d = x.reshape(B * L, D)
    xs = _sc_gather(x2d, sidx)
    xs3 = xs.reshape(B, S, D)

    q, k, v = _qkv(xs3, Wq, Wk, Wv)
    attn = _attention(q, k, v)
    out = _proj(attn, Wo)

    out2d = out.reshape(B * S, D)
    output = _sc_combine(x2d, out2d, sidx, cidx)
    return output.reshape(B, L, D)


# speedup vs baseline: 1.4842x; 1.0051x over previous
import functools
import jax
import jax.numpy as jnp
from jax import lax
from jax.experimental import pallas as pl
from jax.experimental.pallas import tpu as pltpu
from jax.experimental.pallas import tpu_sc as plsc

B, L, D = 2, 2048, 2048
H, DH = 16, 128
S = 1024
NEG = -1e30


_RANK_CHUNK = 256


def _router_kernel(x_ref, wr_ref, br_ref, padj_ref, s_ref, rank_ref):
    b = pl.program_id(0)
    s = jnp.dot(
        x_ref[0].astype(jnp.bfloat16),
        wr_ref[...].astype(jnp.bfloat16),
        preferred_element_type=jnp.float32,
    )[:, 0] + br_ref[0]
    s_ref[0, :] = s

    iota_j = lax.broadcasted_iota(jnp.int32, (1, L), 1).astype(jnp.float32)

    def rank_body(c, _):
        base = c * _RANK_CHUNK
        sv = s_ref[0, pl.ds(base, _RANK_CHUNK)].reshape(_RANK_CHUNK, 1)
        row_i = (lax.broadcasted_iota(jnp.int32, (_RANK_CHUNK, 1), 0)
                 + base).astype(jnp.float32)
        srow = s_ref[0, :].reshape(1, L)
        gt = (srow > sv).astype(jnp.float32)
        tie = jnp.logical_and(srow == sv, iota_j < row_i).astype(jnp.float32)
        rank = jnp.sum(gt + tie, axis=1)
        rank_ref[0, pl.ds(base, _RANK_CHUNK)] = rank
        return _

    lax.fori_loop(0, L // _RANK_CHUNK, rank_body, 0)

    def perm_body(c, _):
        base = c * _RANK_CHUNK
        rv = (lax.broadcasted_iota(jnp.int32, (_RANK_CHUNK, 1), 0)
              + base).astype(jnp.float32)
        rrow = rank_ref[0, :].reshape(1, L)
        eq = (rrow == rv).astype(jnp.float32)
        idx = jnp.sum(eq * iota_j, axis=1)
        padj_ref[0, 0, pl.ds(base, _RANK_CHUNK)] = idx.astype(jnp.int32) + b * L
        return _

    lax.fori_loop(0, L // _RANK_CHUNK, perm_body, 0)


def _router_topk(x, Wr, br):
    return pl.pallas_call(
        _router_kernel,
        grid=(B,),
        in_specs=[
            pl.BlockSpec((1, L, D), lambda b: (b, 0, 0)),
            pl.BlockSpec((D, 1), lambda b: (0, 0)),
            pl.BlockSpec((1,), lambda b: (0,), memory_space=pltpu.SMEM),
        ],
        out_specs=pl.BlockSpec((1, 1, L), lambda b: (b, 0, 0)),
        out_shape=jax.ShapeDtypeStruct((B, 1, L), jnp.int32),
        scratch_shapes=[
            pltpu.VMEM((1, L), jnp.float32),
            pltpu.VMEM((1, L), jnp.float32),
        ],
    )(x, Wr, br)


_QKV_BM = 512
_QKV_BN = 512


def _qkv_kernel(xs_ref, wq_ref, wk_ref, wv_ref, q_ref, k_ref, v_ref):
    xb = xs_ref[0].astype(jnp.bfloat16)
    for w_ref, o_ref in ((wq_ref, q_ref), (wk_ref, k_ref), (wv_ref, v_ref)):
        wb = w_ref[...].astype(jnp.bfloat16)
        acc = jnp.dot(xb, wb, preferred_element_type=jnp.float32)
        o_ref[0] = acc.astype(jnp.bfloat16)


def _qkv(xs3, Wq, Wk, Wv):
    grid = (D // _QKV_BN, B, S // _QKV_BM)
    return pl.pallas_call(
        _qkv_kernel,
        grid=grid,
        in_specs=[
            pl.BlockSpec((1, _QKV_BM, D), lambda n, b, m: (b, m, 0)),
            pl.BlockSpec((D, _QKV_BN), lambda n, b, m: (0, n)),
            pl.BlockSpec((D, _QKV_BN), lambda n, b, m: (0, n)),
            pl.BlockSpec((D, _QKV_BN), lambda n, b, m: (0, n)),
        ],
        out_specs=[
            pl.BlockSpec((1, _QKV_BM, _QKV_BN), lambda n, b, m: (b, m, n)),
            pl.BlockSpec((1, _QKV_BM, _QKV_BN), lambda n, b, m: (b, m, n)),
            pl.BlockSpec((1, _QKV_BM, _QKV_BN), lambda n, b, m: (b, m, n)),
        ],
        out_shape=[jax.ShapeDtypeStruct((B, S, D), jnp.bfloat16)] * 3,
    )(xs3, Wq, Wk, Wv)


_ATT_BQ = 512
_SCALE = DH ** -0.5


def _attn_kernel(q_ref, k_ref, v_ref, o_ref, acc_ref, l_ref):
    qi = pl.program_id(1)
    qpos = lax.broadcasted_iota(jnp.int32, (_ATT_BQ, _ATT_BQ), 0)
    kpos = lax.broadcasted_iota(jnp.int32, (_ATT_BQ, _ATT_BQ), 1)
    diag_bias = jnp.where(qpos >= kpos, 0.0, NEG)
    bias0 = jnp.where(qi == 0, diag_bias, 0.0)
    qb = q_ref[0]
    for h in range(H):
        sl = slice(h * DH, (h + 1) * DH)
        qh = qb[:, sl]

        def chunk(j, bias):
            kh = k_ref[0, pl.ds(j * _ATT_BQ, _ATT_BQ), sl]
            vh = v_ref[0, pl.ds(j * _ATT_BQ, _ATT_BQ), sl]
            s = lax.dot_general(
                qh, kh, (((1,), (1,)), ((), ())),
                preferred_element_type=jnp.float32,
            ) * _SCALE
            p = jnp.exp(s + bias)
            lc = jnp.sum(p, axis=1, keepdims=True)
            oc = jnp.dot(p.astype(jnp.bfloat16), vh,
                         preferred_element_type=jnp.float32)
            return oc, lc

        o0, l0 = chunk(0, bias0)
        acc_ref[...] = o0
        l_ref[...] = l0

        @pl.when(qi > 0)
        def _():
            o1, l1 = chunk(1, diag_bias)
            acc_ref[...] += o1
            l_ref[...] += l1

        o_ref[0, :, sl] = (acc_ref[...] / l_ref[...]).astype(jnp.bfloat16)


def _attention(q, k, v):
    return pl.pallas_call(
        _attn_kernel,
        grid=(B, S // _ATT_BQ),
        in_specs=[
            pl.BlockSpec((1, _ATT_BQ, D), lambda b, m: (b, m, 0)),
            pl.BlockSpec((1, S, D), lambda b, m: (b, 0, 0)),
            pl.BlockSpec((1, S, D), lambda b, m: (b, 0, 0)),
        ],
        out_specs=pl.BlockSpec((1, _ATT_BQ, D), lambda b, m: (b, m, 0)),
        out_shape=jax.ShapeDtypeStruct((B, S, D), jnp.bfloat16),
        scratch_shapes=[
            pltpu.VMEM((_ATT_BQ, DH), jnp.float32),
            pltpu.VMEM((_ATT_BQ, 1), jnp.float32),
        ],
    )(q, k, v)


_PRJ_BM = 512
_PRJ_BN = 1024


def _proj_kernel(a_ref, wo_ref, o_ref):
    wb = wo_ref[...].astype(jnp.bfloat16)
    o_ref[0] = jnp.dot(a_ref[0], wb, preferred_element_type=jnp.float32)


def _proj(attn, Wo):
    grid = (D // _PRJ_BN, B, S // _PRJ_BM)
    return pl.pallas_call(
        _proj_kernel,
        grid=grid,
        in_specs=[
            pl.BlockSpec((1, _PRJ_BM, D), lambda n, b, m: (b, m, 0)),
            pl.BlockSpec((D, _PRJ_BN), lambda n, b, m: (0, n)),
        ],
        out_specs=pl.BlockSpec((1, _PRJ_BM, _PRJ_BN), lambda n, b, m: (b, m, n)),
        out_shape=jax.ShapeDtypeStruct((B, S, D), jnp.float32),
    )(attn, Wo)


_NW = 32
_CH = 16

def _sc_mesh():
    return plsc.VectorSubcoreMesh(core_axis_name="c", subcore_axis_name="s")


def _pipelined_rows(in_hbm, o_hbm, idx_v, rows_v, gsem, osem,
                    nch, src_indexed, dst_indexed, in_base, out_base):
    def src_copy(c):
        if src_indexed:
            return pltpu.make_async_copy(
                in_hbm.at[idx_v.at[c]], rows_v.at[c % 2], gsem)
        return pltpu.make_async_copy(
            in_hbm.at[pl.ds(in_base + c * _CH, _CH)], rows_v.at[c % 2], gsem)

    def dst_copy(c):
        if dst_indexed:
            return pltpu.make_async_copy(
                rows_v.at[c % 2], o_hbm.at[idx_v.at[c]], osem)
        return pltpu.make_async_copy(
            rows_v.at[c % 2], o_hbm.at[pl.ds(out_base + c * _CH, _CH)], osem)

    src_copy(0).start()
    for c in range(nch):
        src_copy(c).wait()
        if c + 1 < nch:
            if c >= 1:
                dst_copy(c - 1).wait()
            src_copy(c + 1).start()
        dst_copy(c).start()
    dst_copy(nch - 1).wait()
    if nch >= 2:
        dst_copy(nch - 2).wait()


def _load_idx(i_hbm, idx_v, base, nch):
    for c in range(nch):
        pltpu.sync_copy(i_hbm.at[pl.ds(base + c * _CH, _CH)], idx_v.at[c])


def _sc_gather(x2d, gidx):
    n = gidx.shape[0]
    rows_w = n // _NW
    nch = rows_w // _CH

    @functools.partial(
        pl.kernel,
        out_type=jax.ShapeDtypeStruct((n, D), jnp.float32),
        mesh=_sc_mesh(),
        scratch_types=[
            pltpu.VMEM((nch, _CH), jnp.int32),
            pltpu.VMEM((2, _CH, D), jnp.float32),
            pltpu.SemaphoreType.DMA,
            pltpu.SemaphoreType.DMA,
        ],
    )
    def k(x_hbm, i_hbm, o_hbm, idx_v, rows_v, gsem, osem):
        wid = lax.axis_index("s") * 2 + lax.axis_index("c")
        base = wid * rows_w
        _load_idx(i_hbm, idx_v, base, nch)
        _pipelined_rows(x_hbm, o_hbm, idx_v, rows_v, gsem, osem,
                        nch, True, False, 0, base)

    return k(x2d, gidx)


def _sc_combine(x2d, out2d, sidx, cidx):
    ns = sidx.shape[0]
    nc = cidx.shape[0]
    ns_w = ns // _NW
    nc_w = nc // _NW
    nch_s = ns_w // _CH
    nch_c = nc_w // _CH

    @functools.partial(
        pl.kernel,
        out_type=jax.ShapeDtypeStruct((B * L, D), jnp.float32),
        mesh=_sc_mesh(),
        scratch_types=[
            pltpu.VMEM((max(nch_s, nch_c), _CH), jnp.int32),
            pltpu.VMEM((2, _CH, D), jnp.float32),
            pltpu.SemaphoreType.DMA,
            pltpu.SemaphoreType.DMA,
        ],
    )
    def k(x_hbm, out_hbm, si_hbm, ci_hbm, o_hbm, idx_v, rows_v, gsem, osem):
        wid = lax.axis_index("s") * 2 + lax.axis_index("c")
        sbase = wid * ns_w
        cbase = wid * nc_w

        _load_idx(si_hbm, idx_v, sbase, nch_s)
        _pipelined_rows(out_hbm, o_hbm, idx_v, rows_v, gsem, osem,
                        nch_s, False, True, sbase, 0)

        _load_idx(ci_hbm, idx_v, cbase, nch_c)
        _pipelined_rows(x_hbm, o_hbm, idx_v, rows_v, gsem, osem,
                        nch_c, True, True, 0, 0)

    return k(x2d, out2d, sidx, cidx)


def kernel(x, Wr, br, Wq, Wk, Wv, Wo):
    padj = _router_topk(x, Wr, br)[:, 0, :]
    sidx = padj[:, :S].reshape(-1)
    cidx = padj[:, S:].reshape(-1)

    x2d = x.reshape(B * L, D)
    xs = _sc_gather(x2d, sidx)
    xs3 = xs.reshape(B, S, D)

    q, k, v = _qkv(xs3, Wq, Wk, Wv)
    attn = _attention(q, k, v)
    out = _proj(attn, Wo)

    out2d = out.reshape(B * S, D)
    output = _sc_combine(x2d, out2d, sidx, cidx)
    return output.reshape(B, L, D)
